# Initial kernel scaffold; baseline (speedup 1.0000x reference)
import jax
import jax.numpy as jnp
from jax.experimental import pallas as pl


def kernel(x, edge_index, W, b):
    raise NotImplementedError("write your pallas kernel here")


# speedup vs baseline: 27.2299x; 27.2299x over previous
import jax
import jax.numpy as jnp
from jax import lax
from jax.experimental import pallas as pl
from jax.experimental.pallas import tpu as pltpu
from jax.experimental.pallas import tpu_sc as plsc

N = 10000
D = 128
E = 320000
NC, NS = 2, 16
NW = NC * NS
EPW = E // NW
K = 80
NCHUNK = EPW // K
NP = 10240
RPT = NP // NS
BLK = 1000

_mesh = plsc.VectorSubcoreMesh(
    core_axis_name="c", subcore_axis_name="s", num_cores=NC, num_subcores=NS
)


def _hist_body(dst_hbm, out_hbm, dstv, degv):
    c = lax.axis_index("c")
    s = lax.axis_index("s")
    wid = c * NS + s
    pltpu.sync_copy(dst_hbm.at[wid], dstv)
    zeros16 = jnp.zeros((16,), jnp.float32)

    def zero(i, carry):
        degv[pl.ds(i * 16, 16)] = zeros16
        return carry

    lax.fori_loop(0, N // 16, zero, 0)
    ones16 = jnp.ones((16,), jnp.float32)

    def add(i, carry):
        idx = dstv[pl.ds(i * 16, 16)]
        plsc.addupdate_scatter(degv, [idx], ones16)
        return carry

    lax.fori_loop(0, EPW // 16, add, 0)
    pltpu.sync_copy(degv, out_hbm.at[wid])


_hist = pl.kernel(
    _hist_body,
    out_type=jax.ShapeDtypeStruct((NW, N), jnp.float32),
    mesh=_mesh,
    compiler_params=pltpu.CompilerParams(needs_layout_passes=False),
    scratch_types=[
        pltpu.VMEM((EPW,), jnp.int32),
        pltpu.VMEM((N,), jnp.float32),
    ],
)


def _agg_body(hp_hbm, src_hbm, dst_hbm, zero_hbm, out_hbm, srcv, dstv, rows, acc, sem):
    c = lax.axis_index("c")
    s = lax.axis_index("s")
    wid = c * NS + s
    pltpu.sync_copy(src_hbm.at[wid], srcv)
    pltpu.sync_copy(dst_hbm.at[wid], dstv)
    pltpu.sync_copy(zero_hbm, acc.at[pl.ds(s * RPT, RPT)])
    plsc.subcore_barrier()

    def step(j, carry):
        pltpu.async_copy(hp_hbm.at[srcv.at[j]], rows, sem).wait()
        pltpu.sync_copy(rows, acc.at[dstv.at[j]], add=True)
        return carry

    lax.fori_loop(0, NCHUNK, step, 0)
    plsc.subcore_barrier()
    pltpu.sync_copy(acc.at[pl.ds(s * RPT, RPT)], out_hbm.at[c, pl.ds(s * RPT, RPT)])


_agg = pl.kernel(
    _agg_body,
    out_type=jax.ShapeDtypeStruct((NC, NP, D), jnp.float32),
    mesh=_mesh,
    compiler_params=pltpu.CompilerParams(needs_layout_passes=False),
    scratch_types=[
        pltpu.VMEM((NCHUNK, K), jnp.int32),
        pltpu.VMEM((NCHUNK, K), jnp.int32),
        pltpu.VMEM((K, D), jnp.float32),
        pltpu.VMEM_SHARED((NP, D), jnp.float32),
        pltpu.SemaphoreType.DMA,
    ],
)


def _mm_body(pdt_ref, x_ref, w_ref, hp_ref):
    deg = jnp.sum(pdt_ref[...], axis=1) + 1.0
    dis = lax.rsqrt(deg)
    h = jnp.dot(x_ref[...], w_ref[...], preferred_element_type=jnp.float32)
    hp_ref[...] = h * dis[:, None]


_mm = pl.pallas_call(
    _mm_body,
    grid=(N // BLK,),
    in_specs=[
        pl.BlockSpec((BLK, NW), lambda i: (i, 0)),
        pl.BlockSpec((BLK, D), lambda i: (i, 0)),
        pl.BlockSpec((D, D), lambda i: (0, 0)),
    ],
    out_specs=pl.BlockSpec((BLK, D), lambda i: (i, 0)),
    out_shape=jax.ShapeDtypeStruct((N, D), jnp.float32),
)


def _fin_body(pdt_ref, p_ref, hp_ref, b_ref, out_ref):
    deg = jnp.sum(pdt_ref[...], axis=1) + 1.0
    dis = lax.rsqrt(deg)
    p = p_ref[...]
    v = (p[0] + p[1] + hp_ref[...]) * dis[:, None] + b_ref[...]
    act = jnp.maximum(v, 0.0)
    m = jnp.max(act, axis=1, keepdims=True)
    ii = lax.broadcasted_iota(jnp.int32, act.shape, 1)
    out_ref[...] = jnp.min(jnp.where(act >= m, ii, D), axis=1)[:, None]


_fin = pl.pallas_call(
    _fin_body,
    grid=(N // BLK,),
    in_specs=[
        pl.BlockSpec((BLK, NW), lambda i: (i, 0)),
        pl.BlockSpec((NC, BLK, D), lambda i: (0, i, 0)),
        pl.BlockSpec((BLK, D), lambda i: (i, 0)),
        pl.BlockSpec((1, D), lambda i: (0, 0)),
    ],
    out_specs=pl.BlockSpec((BLK, 1), lambda i: (i, 0)),
    out_shape=jax.ShapeDtypeStruct((N, 1), jnp.int32),
)


def kernel(x, edge_index, W, b):
    src = edge_index[0].astype(jnp.int32)
    dst = edge_index[1].astype(jnp.int32)
    pd = _hist(dst.reshape(NW, EPW))
    pdt = pd.T
    hp = _mm(pdt, x, W)
    zero = jnp.zeros((RPT, D), jnp.float32)
    parts = _agg(hp, src.reshape(NW, NCHUNK, K), dst.reshape(NW, NCHUNK, K), zero)
    out2 = _fin(pdt, parts, hp, b.reshape(1, D))
    return out2.reshape(N)


# speedup vs baseline: 39.3504x; 1.4451x over previous
import jax
import jax.numpy as jnp
from jax import lax
from jax.experimental import pallas as pl
from jax.experimental.pallas import tpu as pltpu
from jax.experimental.pallas import tpu_sc as plsc

N = 10000
D = 128
E = 320000
NC, NS = 2, 16
NW = NC * NS
EPW = E // NW
K = 80
NCHUNK = EPW // K
NP = 10240
RPT = NP // NS
BLK = 1000

_mesh = plsc.VectorSubcoreMesh(
    core_axis_name="c", subcore_axis_name="s", num_cores=NC, num_subcores=NS
)


def _hist_body(dst_hbm, out_hbm, dstv, degv):
    c = lax.axis_index("c")
    s = lax.axis_index("s")
    wid = c * NS + s
    pltpu.sync_copy(dst_hbm.at[wid], dstv)
    zeros16 = jnp.zeros((16,), jnp.float32)

    def zero(i, carry):
        degv[pl.ds(i * 16, 16)] = zeros16
        return carry

    lax.fori_loop(0, N // 16, zero, 0)
    ones16 = jnp.ones((16,), jnp.float32)

    def add(i, carry):
        idx = dstv[pl.ds(i * 16, 16)]
        plsc.addupdate_scatter(degv, [idx], ones16)
        return carry

    lax.fori_loop(0, EPW // 16, add, 0)
    pltpu.sync_copy(degv, out_hbm.at[wid])


_hist = pl.kernel(
    _hist_body,
    out_type=jax.ShapeDtypeStruct((NW, N), jnp.float32),
    mesh=_mesh,
    compiler_params=pltpu.CompilerParams(needs_layout_passes=False),
    scratch_types=[
        pltpu.VMEM((EPW,), jnp.int32),
        pltpu.VMEM((N,), jnp.float32),
    ],
)


NBUF = 1
NGROUP = (NCHUNK + NBUF - 1) // NBUF


def _agg_body(hp_hbm, src_hbm, dst_hbm, zero_hbm, out_hbm, srcv, dstv, *rest):
    rows = rest[: 2 * NBUF]
    acc = rest[2 * NBUF]
    sems = rest[2 * NBUF + 1:]
    c = lax.axis_index("c")
    s = lax.axis_index("s")
    wid = c * NS + s
    pltpu.sync_copy(src_hbm.at[wid], srcv)
    pltpu.sync_copy(dst_hbm.at[wid], dstv)
    pltpu.sync_copy(zero_hbm, acc.at[pl.ds(s * RPT, RPT)])
    plsc.subcore_barrier()

    def fire(g, bank):
        for t in range(NBUF):
            j = g * NBUF + t

            @pl.when(j < NCHUNK)
            def _():
                pltpu.async_copy(
                    hp_hbm.at[srcv.at[pl.ds(j * K, K)]],
                    rows[bank * NBUF + t],
                    sems[bank * NBUF + t],
                )

    def drain(g, bank):
        for t in range(NBUF):
            j = g * NBUF + t

            @pl.when(j < NCHUNK)
            def _():
                pltpu.make_async_copy(
                    hp_hbm.at[srcv.at[pl.ds(j * K, K)]],
                    rows[bank * NBUF + t],
                    sems[bank * NBUF + t],
                ).wait()
                pltpu.sync_copy(rows[bank * NBUF + t], acc.at[dstv.at[j]], add=True)

    fire(0, 0)

    def step(h, carry):
        ga = 2 * h
        gb = 2 * h + 1
        fire(gb, 1)
        drain(ga, 0)
        fire(ga + 2, 0)
        drain(gb, 1)
        return carry

    lax.fori_loop(0, (NGROUP + 1) // 2, step, 0)
    plsc.subcore_barrier()
    pltpu.sync_copy(acc.at[pl.ds(s * RPT, RPT)], out_hbm.at[c, pl.ds(s * RPT, RPT)])


_agg = pl.kernel(
    _agg_body,
    out_type=jax.ShapeDtypeStruct((NC, NP, D), jnp.float32),
    mesh=_mesh,
    compiler_params=pltpu.CompilerParams(needs_layout_passes=False),
    scratch_types=[
        pltpu.VMEM((EPW,), jnp.int32),
        pltpu.VMEM((NCHUNK, K), jnp.int32),
    ]
    + [pltpu.VMEM((K, D), jnp.float32)] * (2 * NBUF)
    + [pltpu.VMEM_SHARED((NP, D), jnp.float32)]
    + [pltpu.SemaphoreType.DMA] * (2 * NBUF),
)


def _mm_body(pdt_ref, x_ref, w_ref, hp_ref):
    deg = jnp.sum(pdt_ref[...], axis=1) + 1.0
    dis = lax.rsqrt(deg)
    h = jnp.dot(x_ref[...], w_ref[...], preferred_element_type=jnp.float32)
    hp_ref[...] = h * dis[:, None]


_mm = pl.pallas_call(
    _mm_body,
    grid=(N // BLK,),
    in_specs=[
        pl.BlockSpec((BLK, NW), lambda i: (i, 0)),
        pl.BlockSpec((BLK, D), lambda i: (i, 0)),
        pl.BlockSpec((D, D), lambda i: (0, 0)),
    ],
    out_specs=pl.BlockSpec((BLK, D), lambda i: (i, 0)),
    out_shape=jax.ShapeDtypeStruct((N, D), jnp.float32),
)


def _fin_body(pdt_ref, p_ref, hp_ref, b_ref, out_ref):
    deg = jnp.sum(pdt_ref[...], axis=1) + 1.0
    dis = lax.rsqrt(deg)
    p = p_ref[...]
    v = (p[0] + p[1] + hp_ref[...]) * dis[:, None] + b_ref[...]
    act = jnp.maximum(v, 0.0)
    m = jnp.max(act, axis=1, keepdims=True)
    ii = lax.broadcasted_iota(jnp.int32, act.shape, 1)
    out_ref[...] = jnp.min(jnp.where(act >= m, ii, D), axis=1)[:, None]


_fin = pl.pallas_call(
    _fin_body,
    grid=(N // BLK,),
    in_specs=[
        pl.BlockSpec((BLK, NW), lambda i: (i, 0)),
        pl.BlockSpec((NC, BLK, D), lambda i: (0, i, 0)),
        pl.BlockSpec((BLK, D), lambda i: (i, 0)),
        pl.BlockSpec((1, D), lambda i: (0, 0)),
    ],
    out_specs=pl.BlockSpec((BLK, 1), lambda i: (i, 0)),
    out_shape=jax.ShapeDtypeStruct((N, 1), jnp.int32),
)


def kernel(x, edge_index, W, b):
    src = edge_index[0].astype(jnp.int32)
    dst = edge_index[1].astype(jnp.int32)
    pd = _hist(dst.reshape(NW, EPW))
    pdt = pd.T
    hp = _mm(pdt, x, W)
    zero = jnp.zeros((RPT, D), jnp.float32)
    parts = _agg(hp, src.reshape(NW, EPW), dst.reshape(NW, NCHUNK, K), zero)
    out2 = _fin(pdt, parts, hp, b.reshape(1, D))
    return out2.reshape(N)


# speedup vs baseline: 44.6110x; 1.1337x over previous
import jax
import jax.numpy as jnp
from jax import lax
from jax.experimental import pallas as pl
from jax.experimental.pallas import tpu as pltpu
from jax.experimental.pallas import tpu_sc as plsc

N = 10000
D = 128
E = 320000
NC, NS = 2, 16
NW = NC * NS
EPW = E // NW
K = 40
NCHUNK = EPW // K
NP = 10240
RPT = NP // NS
BLK = 1000
NBANK = 4

_mesh = plsc.VectorSubcoreMesh(
    core_axis_name="c", subcore_axis_name="s", num_cores=NC, num_subcores=NS
)


def _hist_body(dst_hbm, out_hbm, dstv, degv):
    c = lax.axis_index("c")
    s = lax.axis_index("s")
    wid = c * NS + s
    pltpu.sync_copy(dst_hbm.at[wid], dstv)
    zeros16 = jnp.zeros((16,), jnp.float32)

    def zero(i, carry):
        degv[pl.ds(i * 16, 16)] = zeros16
        return carry

    lax.fori_loop(0, N // 16, zero, 0)
    ones16 = jnp.ones((16,), jnp.float32)

    def add(i, carry):
        idx = dstv[pl.ds(i * 16, 16)]
        plsc.addupdate_scatter(degv, [idx], ones16)
        return carry

    lax.fori_loop(0, EPW // 16, add, 0)
    pltpu.sync_copy(degv, out_hbm.at[wid])


_hist = pl.kernel(
    _hist_body,
    out_type=jax.ShapeDtypeStruct((NW, N), jnp.float32),
    mesh=_mesh,
    compiler_params=pltpu.CompilerParams(needs_layout_passes=False),
    scratch_types=[
        pltpu.VMEM((EPW,), jnp.int32),
        pltpu.VMEM((N,), jnp.float32),
    ],
)


def _agg_body(hp_hbm, src_hbm, dst_hbm, zero_hbm, out_hbm, srcv, dstv, *rest):
    rows = rest[:NBANK]
    acc = rest[NBANK]
    gsems = rest[NBANK + 1:]
    c = lax.axis_index("c")
    s = lax.axis_index("s")
    wid = c * NS + s
    pltpu.sync_copy(src_hbm.at[wid], srcv)
    pltpu.sync_copy(dst_hbm.at[wid], dstv)
    pltpu.sync_copy(zero_hbm, acc.at[pl.ds(s * RPT, RPT)])

    def fire(j, t):
        @pl.when(j < NCHUNK)
        def _():
            pltpu.async_copy(hp_hbm.at[srcv.at[pl.ds(j * K, K)]], rows[t], gsems[t])

    def drain(j, t):
        @pl.when(j < NCHUNK)
        def _():
            pltpu.make_async_copy(
                hp_hbm.at[srcv.at[pl.ds(j * K, K)]], rows[t], gsems[t]
            ).wait()
            pltpu.sync_copy(rows[t], acc.at[dstv.at[pl.ds(j * K, K)]], add=True)

    for t in range(NBANK):
        fire(t, t)
    plsc.subcore_barrier()

    def round_(r, carry):
        for t in range(NBANK):
            j = r * NBANK + t
            drain(j, t)
            fire(j + NBANK, t)
        return carry

    lax.fori_loop(0, (NCHUNK + NBANK - 1) // NBANK, round_, 0)
    plsc.subcore_barrier()
    pltpu.sync_copy(acc.at[pl.ds(s * RPT, RPT)], out_hbm.at[c, pl.ds(s * RPT, RPT)])


_agg = pl.kernel(
    _agg_body,
    out_type=jax.ShapeDtypeStruct((NC, NP, D), jnp.float32),
    mesh=_mesh,
    compiler_params=pltpu.CompilerParams(needs_layout_passes=False),
    scratch_types=[
        pltpu.VMEM((EPW,), jnp.int32),
        pltpu.VMEM((EPW,), jnp.int32),
    ]
    + [pltpu.VMEM((K, D), jnp.float32)] * NBANK
    + [pltpu.VMEM_SHARED((NP, D), jnp.float32)]
    + [pltpu.SemaphoreType.DMA] * NBANK,
)


def _mm_body(pds_ref, x_ref, w_ref, hp_ref):
    deg = pds_ref[0, 0] + 1.0
    dis = lax.rsqrt(deg)
    h = jnp.dot(x_ref[...], w_ref[...], preferred_element_type=jnp.float32)
    hp_ref[...] = h * dis[:, None]


_mm = pl.pallas_call(
    _mm_body,
    grid=(N // BLK,),
    in_specs=[
        pl.BlockSpec((1, 1, BLK), lambda i: (i, 0, 0)),
        pl.BlockSpec((BLK, D), lambda i: (i, 0)),
        pl.BlockSpec((D, D), lambda i: (0, 0)),
    ],
    out_specs=pl.BlockSpec((BLK, D), lambda i: (i, 0)),
    out_shape=jax.ShapeDtypeStruct((N, D), jnp.float32),
)


def _fin_body(pds_ref, p_ref, hp_ref, b_ref, out_ref):
    deg = pds_ref[0, 0] + 1.0
    dis = lax.rsqrt(deg)
    p = p_ref[...]
    v = (p[0] + p[1] + hp_ref[...]) * dis[:, None] + b_ref[...]
    act = jnp.maximum(v, 0.0)
    m = jnp.max(act, axis=1, keepdims=True)
    ii = lax.broadcasted_iota(jnp.int32, act.shape, 1)
    out_ref[...] = jnp.min(jnp.where(act >= m, ii, D), axis=1)[:, None]


_fin = pl.pallas_call(
    _fin_body,
    grid=(N // BLK,),
    in_specs=[
        pl.BlockSpec((1, 1, BLK), lambda i: (i, 0, 0)),
        pl.BlockSpec((NC, BLK, D), lambda i: (0, i, 0)),
        pl.BlockSpec((BLK, D), lambda i: (i, 0)),
        pl.BlockSpec((1, D), lambda i: (0, 0)),
    ],
    out_specs=pl.BlockSpec((BLK, 1), lambda i: (i, 0)),
    out_shape=jax.ShapeDtypeStruct((N, 1), jnp.int32),
)


def kernel(x, edge_index, W, b):
    src = edge_index[0].astype(jnp.int32).reshape(NW, EPW)
    dst = edge_index[1].astype(jnp.int32).reshape(NW, EPW)
    pd = _hist(dst)
    pds = pd.sum(axis=0).reshape(N // BLK, 1, BLK)
    hp = _mm(pds, x, W)
    zero = jnp.zeros((RPT, D), jnp.float32)
    parts = _agg(hp, src, dst, zero)
    out2 = _fin(pds, parts, hp, b.reshape(1, D))
    return out2.reshape(N)


# speedup vs baseline: 44.9738x; 1.0081x over previous
import jax
import jax.numpy as jnp
from jax import lax
from jax.experimental import pallas as pl
from jax.experimental.pallas import tpu as pltpu
from jax.experimental.pallas import tpu_sc as plsc

N = 10000
D = 128
E = 320000
NC, NS = 2, 16
NW = NC * NS
EPW = E // NW
K = 40
NCHUNK = EPW // K
NP = 10240
RPT = NP // NS
BLK = 1024
NBANK = 4

_mesh = plsc.VectorSubcoreMesh(
    core_axis_name="c", subcore_axis_name="s", num_cores=NC, num_subcores=NS
)


def _hist_body(dst_hbm, out_hbm, dstv, degv):
    c = lax.axis_index("c")
    s = lax.axis_index("s")
    wid = c * NS + s
    pltpu.sync_copy(dst_hbm.at[wid], dstv)
    zeros16 = jnp.zeros((16,), jnp.float32)

    def zero(i, carry):
        degv[pl.ds(i * 16, 16)] = zeros16
        return carry

    lax.fori_loop(0, NP // 16, zero, 0)
    ones16 = jnp.ones((16,), jnp.float32)

    def add(i, carry):
        idx = dstv[pl.ds(i * 16, 16)]
        plsc.addupdate_scatter(degv, [idx], ones16)
        return carry

    lax.fori_loop(0, EPW // 16, add, 0)
    pltpu.sync_copy(degv, out_hbm.at[wid])


_hist = pl.kernel(
    _hist_body,
    out_type=jax.ShapeDtypeStruct((NW, NP), jnp.float32),
    mesh=_mesh,
    compiler_params=pltpu.CompilerParams(needs_layout_passes=False),
    scratch_types=[
        pltpu.VMEM((EPW,), jnp.int32),
        pltpu.VMEM((NP,), jnp.float32),
    ],
)


def _agg_body(hp_hbm, src_hbm, dst_hbm, zero_hbm, out_hbm, srcv, dstv, *rest):
    rows = rest[:NBANK]
    acc = rest[NBANK]
    gsems = rest[NBANK + 1:]
    c = lax.axis_index("c")
    s = lax.axis_index("s")
    wid = c * NS + s
    pltpu.sync_copy(src_hbm.at[wid], srcv)
    pltpu.sync_copy(dst_hbm.at[wid], dstv)
    pltpu.sync_copy(zero_hbm, acc.at[pl.ds(s * RPT, RPT)])

    def fire(j, t):
        @pl.when(j < NCHUNK)
        def _():
            pltpu.async_copy(hp_hbm.at[srcv.at[pl.ds(j * K, K)]], rows[t], gsems[t])

    def drain(j, t):
        @pl.when(j < NCHUNK)
        def _():
            pltpu.make_async_copy(
                hp_hbm.at[srcv.at[pl.ds(j * K, K)]], rows[t], gsems[t]
            ).wait()
            pltpu.sync_copy(rows[t], acc.at[dstv.at[pl.ds(j * K, K)]], add=True)

    for t in range(NBANK):
        fire(t, t)
    plsc.subcore_barrier()

    def round_(r, carry):
        for t in range(NBANK):
            j = r * NBANK + t
            drain(j, t)
            fire(j + NBANK, t)
        return carry

    lax.fori_loop(0, (NCHUNK + NBANK - 1) // NBANK, round_, 0)
    plsc.subcore_barrier()
    pltpu.sync_copy(acc.at[pl.ds(s * RPT, RPT)], out_hbm.at[c, pl.ds(s * RPT, RPT)])


_agg = pl.kernel(
    _agg_body,
    out_type=jax.ShapeDtypeStruct((NC, NP, D), jnp.float32),
    mesh=_mesh,
    compiler_params=pltpu.CompilerParams(needs_layout_passes=False),
    scratch_types=[
        pltpu.VMEM((EPW,), jnp.int32),
        pltpu.VMEM((EPW,), jnp.int32),
    ]
    + [pltpu.VMEM((K, D), jnp.float32)] * NBANK
    + [pltpu.VMEM_SHARED((NP, D), jnp.float32)]
    + [pltpu.SemaphoreType.DMA] * NBANK,
)


def _mm_body(pd_ref, x_ref, w_ref, hp_ref):
    i = pl.program_id(0)
    deg = jnp.sum(pd_ref[:, pl.ds(i * BLK, BLK)], axis=0) + 1.0
    dis = lax.rsqrt(deg)
    h = jnp.dot(x_ref[...], w_ref[...], preferred_element_type=jnp.float32)
    hp_ref[...] = h * dis[:, None]


_mm = pl.pallas_call(
    _mm_body,
    grid=(NP // BLK,),
    in_specs=[
        pl.BlockSpec((NW, NP), lambda i: (0, 0)),
        pl.BlockSpec((BLK, D), lambda i: (i, 0)),
        pl.BlockSpec((D, D), lambda i: (0, 0)),
    ],
    out_specs=pl.BlockSpec((BLK, D), lambda i: (i, 0)),
    out_shape=jax.ShapeDtypeStruct((NP, D), jnp.float32),
)


def _fin_body(pd_ref, p_ref, hp_ref, b_ref, out_ref):
    i = pl.program_id(0)
    deg = jnp.sum(pd_ref[:, pl.ds(i * BLK, BLK)], axis=0) + 1.0
    dis = lax.rsqrt(deg)
    p = p_ref[...]
    v = (p[0] + p[1] + hp_ref[...]) * dis[:, None] + b_ref[...]
    act = jnp.maximum(v, 0.0)
    m = jnp.max(act, axis=1, keepdims=True)
    ii = lax.broadcasted_iota(jnp.int32, act.shape, 1)
    out_ref[...] = jnp.min(jnp.where(act >= m, ii, D), axis=1)[:, None]


_fin = pl.pallas_call(
    _fin_body,
    grid=(NP // BLK,),
    in_specs=[
        pl.BlockSpec((NW, NP), lambda i: (0, 0)),
        pl.BlockSpec((NC, BLK, D), lambda i: (0, i, 0)),
        pl.BlockSpec((BLK, D), lambda i: (i, 0)),
        pl.BlockSpec((1, D), lambda i: (0, 0)),
    ],
    out_specs=pl.BlockSpec((BLK, 1), lambda i: (i, 0)),
    out_shape=jax.ShapeDtypeStruct((N, 1), jnp.int32),
)


def kernel(x, edge_index, W, b):
    src = edge_index[0].astype(jnp.int32).reshape(NW, EPW)
    dst = edge_index[1].astype(jnp.int32).reshape(NW, EPW)
    pd = _hist(dst)
    hp = _mm(pd, x, W)
    zero = jnp.zeros((RPT, D), jnp.float32)
    parts = _agg(hp, src, dst, zero)
    out2 = _fin(pd, parts, hp, b.reshape(1, D))
    return out2.reshape(N)
